# Initial kernel scaffold; baseline (speedup 1.0000x reference)
import jax
import jax.numpy as jnp
from jax.experimental import pallas as pl


def kernel(x, edge_index, edge_weight, W_cheb, b_cheb, W_time, b_time, W_res, b_res, ln_gamma, ln_beta, W_final, b_final):
    raise NotImplementedError("write your pallas kernel here")


# speedup vs baseline: 1.6305x; 1.6305x over previous
import functools

import jax
import jax.numpy as jnp
from jax.experimental import pallas as pl

_BLK = 4096


def _body(x_ref, wc_ref, wr_ref, s_ref, o_ref):
    xb = x_ref[0]
    wc = wc_ref[...]
    wr = wr_ref[...]
    b_cheb = s_ref[0:1, 0:1]
    wt = s_ref[0:1, 1:2]
    b_time = s_ref[0:1, 2:3]
    b_res = s_ref[0:1, 3:4]
    gamma = s_ref[0:1, 4:5]
    beta = s_ref[0:1, 5:6]
    wf = s_ref[0:1, 6:7]
    bf = s_ref[0:1, 7:8]

    s1 = jnp.maximum(jnp.sum(xb * wc, axis=1, keepdims=True) + b_cheb, 0.0)
    xt = s1 * wt + b_time
    xr = jnp.sum(xb * wr, axis=1, keepdims=True) + b_res
    z = jnp.maximum(xr + xt, 0.0)
    mu = z
    zc = z - mu
    var = zc * zc
    zn = zc * jax.lax.rsqrt(var + 1e-5) * gamma + beta
    o_ref[...] = zn * wf + bf


@functools.partial(jax.jit, static_argnames=())
def _run(x, W_cheb, b_cheb, W_time, b_time, W_res, b_res,
         ln_gamma, ln_beta, W_final, b_final):
    _, n, f_in, _ = x.shape
    x3 = x.reshape(x.shape[0], n, f_in)
    wc = W_cheb[:, 0][None, :]
    wr = W_res[0, :, 0, 0][None, :]
    scal = jnp.stack([
        b_cheb[0], W_time[0, 0, 0, 1], b_time[0], b_res[0],
        ln_gamma[0], ln_beta[0], W_final[0, 0, 0, 0], b_final[0],
    ])[None, :]

    grid = (pl.cdiv(n, _BLK),)
    out = pl.pallas_call(
        _body,
        grid=grid,
        in_specs=[
            pl.BlockSpec((1, _BLK, f_in), lambda i: (0, i, 0)),
            pl.BlockSpec((1, f_in), lambda i: (0, 0)),
            pl.BlockSpec((1, f_in), lambda i: (0, 0)),
            pl.BlockSpec((1, 8), lambda i: (0, 0)),
        ],
        out_specs=pl.BlockSpec((_BLK, 1), lambda i: (i, 0)),
        out_shape=jax.ShapeDtypeStruct((n, 1), jnp.float32),
    )(x3, wc, wr, scal)
    return out


def kernel(x, edge_index, edge_weight, W_cheb, b_cheb, W_time, b_time,
           W_res, b_res, ln_gamma, ln_beta, W_final, b_final):
    del edge_index, edge_weight
    out = _run(x, W_cheb, b_cheb, W_time, b_time, W_res, b_res,
               ln_gamma, ln_beta, W_final, b_final)
    return (out,)


# speedup vs baseline: 2.1371x; 1.3107x over previous
import jax
import jax.numpy as jnp
from jax.experimental import pallas as pl

_BLK = 4096


def _body(x_ref, w_ref, s_ref, o_ref):
    xb = x_ref[0]
    w = w_ref[...]
    S = jax.lax.dot_general(w, xb, (((1,), (1,)), ((), ())),
                            preferred_element_type=jnp.float32)
    s1d = S[0:1, :]
    xrd = S[1:2, :]

    b_cheb = s_ref[0:1, 0:1]
    wt = s_ref[0:1, 1:2]
    b_time = s_ref[0:1, 2:3]
    b_res = s_ref[0:1, 3:4]
    gamma = s_ref[0:1, 4:5]
    beta = s_ref[0:1, 5:6]
    wf = s_ref[0:1, 6:7]
    bf = s_ref[0:1, 7:8]

    s1 = jnp.maximum(s1d + b_cheb, 0.0)
    xt = s1 * wt + b_time
    xr = xrd + b_res
    z = jnp.maximum(xr + xt, 0.0)
    mu = z
    zc = z - mu
    var = zc * zc
    zn = zc * jax.lax.rsqrt(var + 1e-5) * gamma + beta
    o_ref[...] = zn * wf + bf


@jax.jit
def _run(x, W_cheb, b_cheb, W_time, b_time, W_res, b_res,
         ln_gamma, ln_beta, W_final, b_final):
    _, n, f_in, _ = x.shape
    x3 = x.reshape(x.shape[0], n, f_in)
    w = jnp.zeros((8, f_in), jnp.float32)
    w = w.at[0].set(W_cheb[:, 0]).at[1].set(W_res[0, :, 0, 0])
    scal = jnp.stack([
        b_cheb[0], W_time[0, 0, 0, 1], b_time[0], b_res[0],
        ln_gamma[0], ln_beta[0], W_final[0, 0, 0, 0], b_final[0],
    ])[None, :]

    grid = (pl.cdiv(n, _BLK),)
    out = pl.pallas_call(
        _body,
        grid=grid,
        in_specs=[
            pl.BlockSpec((1, _BLK, f_in), lambda i: (0, i, 0)),
            pl.BlockSpec((8, f_in), lambda i: (0, 0)),
            pl.BlockSpec((1, 8), lambda i: (0, 0)),
        ],
        out_specs=pl.BlockSpec((1, _BLK), lambda i: (0, i)),
        out_shape=jax.ShapeDtypeStruct((1, n), jnp.float32),
    )(x3, w, scal)
    return out.reshape(n, 1)


def kernel(x, edge_index, edge_weight, W_cheb, b_cheb, W_time, b_time,
           W_res, b_res, ln_gamma, ln_beta, W_final, b_final):
    del edge_index, edge_weight
    out = _run(x, W_cheb, b_cheb, W_time, b_time, W_res, b_res,
               ln_gamma, ln_beta, W_final, b_final)
    return (out,)


# speedup vs baseline: 8.8806x; 4.1554x over previous
import functools

import jax
import jax.numpy as jnp
from jax.experimental import pallas as pl
from jax.experimental.pallas import tpu as pltpu

_BLK = 4096


def _epilogue(w, s_ref, xb, size, off, o_ref):
    S = jax.lax.dot_general(w, xb, (((1,), (1,)), ((), ())),
                            preferred_element_type=jnp.float32)
    s1d = S[0:1, :]
    xrd = S[1:2, :]
    b_cheb = s_ref[0:1, 0:1]
    wt = s_ref[0:1, 1:2]
    b_time = s_ref[0:1, 2:3]
    b_res = s_ref[0:1, 3:4]
    gamma = s_ref[0:1, 4:5]
    beta = s_ref[0:1, 5:6]
    wf = s_ref[0:1, 6:7]
    bf = s_ref[0:1, 7:8]
    s1 = jnp.maximum(s1d + b_cheb, 0.0)
    xt = s1 * wt + b_time
    xr = xrd + b_res
    z = jnp.maximum(xr + xt, 0.0)
    mu = z
    zc = z - mu
    var = zc * zc
    zn = zc * jax.lax.rsqrt(var + 1e-5) * gamma + beta
    o_ref[0:1, pl.ds(off, size)] = zn * wf + bf


def _make_body(n):
    nfull = n // _BLK
    tail = n - nfull * _BLK

    def body(x_hbm, w_ref, s_ref, o_ref, buf0, buf1, tbuf, sem0, sem1, semt):
        bufs = (buf0, buf1)
        sems = (sem0, sem1)

        def cp(i):
            return pltpu.make_async_copy(
                x_hbm.at[0, pl.ds(i * _BLK, _BLK), 0, :],
                bufs[i % 2], sems[i % 2])

        def cpt():
            return pltpu.make_async_copy(
                x_hbm.at[0, pl.ds(nfull * _BLK, tail), 0, :], tbuf, semt)

        if nfull > 0:
            cp(0).start()
        if nfull > 1:
            cp(1).start()
        if tail:
            cpt().start()

        w = w_ref[...]
        for i in range(nfull):
            cp(i).wait()
            _epilogue(w, s_ref, bufs[i % 2][...], _BLK, i * _BLK, o_ref)
            if i + 2 < nfull:
                cp(i + 2).start()
        if tail:
            cpt().wait()
            _epilogue(w, s_ref, tbuf[...], tail, nfull * _BLK, o_ref)

    return body


@jax.jit
def _run(x, W_cheb, b_cheb, W_time, b_time, W_res, b_res,
         ln_gamma, ln_beta, W_final, b_final):
    _, n, f_in, _ = x.shape
    x4 = x.reshape(x.shape[0], n, 1, f_in)
    w = jnp.zeros((8, f_in), jnp.float32)
    w = w.at[0].set(W_cheb[:, 0]).at[1].set(W_res[0, :, 0, 0])
    scal = jnp.stack([
        b_cheb[0], W_time[0, 0, 0, 1], b_time[0], b_res[0],
        ln_gamma[0], ln_beta[0], W_final[0, 0, 0, 0], b_final[0],
    ])[None, :]

    tail = n - (n // _BLK) * _BLK
    out = pl.pallas_call(
        _make_body(n),
        in_specs=[
            pl.BlockSpec(memory_space=pl.ANY),
            pl.BlockSpec(memory_space=pltpu.MemorySpace.VMEM),
            pl.BlockSpec(memory_space=pltpu.MemorySpace.VMEM),
        ],
        out_specs=pl.BlockSpec(memory_space=pltpu.MemorySpace.VMEM),
        out_shape=jax.ShapeDtypeStruct((1, n), jnp.float32),
        scratch_shapes=[
            pltpu.VMEM((_BLK, f_in), jnp.float32),
            pltpu.VMEM((_BLK, f_in), jnp.float32),
            pltpu.VMEM((max(tail, 8), f_in), jnp.float32),
            pltpu.SemaphoreType.DMA,
            pltpu.SemaphoreType.DMA,
            pltpu.SemaphoreType.DMA,
        ],
    )(x4, w, scal)
    return out.reshape(n, 1)


def kernel(x, edge_index, edge_weight, W_cheb, b_cheb, W_time, b_time,
           W_res, b_res, ln_gamma, ln_beta, W_final, b_final):
    del edge_index, edge_weight
    out = _run(x, W_cheb, b_cheb, W_time, b_time, W_res, b_res,
               ln_gamma, ln_beta, W_final, b_final)
    return (out,)


# speedup vs baseline: 9.8787x; 1.1124x over previous
import functools

import jax
import jax.numpy as jnp
from jax.experimental import pallas as pl
from jax.experimental.pallas import tpu as pltpu

_BLK = 4096
_NBUF = 4


def _epilogue(w, s_ref, xb, size, off, o_ref):
    S = jax.lax.dot_general(w, xb, (((1,), (1,)), ((), ())),
                            preferred_element_type=jnp.float32)
    s1d = S[0:1, :]
    xrd = S[1:2, :]
    b_cheb = s_ref[0:1, 0:1]
    wt = s_ref[0:1, 1:2]
    b_time = s_ref[0:1, 2:3]
    b_res = s_ref[0:1, 3:4]
    gamma = s_ref[0:1, 4:5]
    beta = s_ref[0:1, 5:6]
    wf = s_ref[0:1, 6:7]
    bf = s_ref[0:1, 7:8]
    s1 = jnp.maximum(s1d + b_cheb, 0.0)
    xt = s1 * wt + b_time
    xr = xrd + b_res
    z = jnp.maximum(xr + xt, 0.0)
    mu = z
    zc = z - mu
    var = zc * zc
    zn = zc * jax.lax.rsqrt(var + 1e-5) * gamma + beta
    o_ref[0:1, pl.ds(off, size)] = zn * wf + bf


def _make_body(n):
    nfull = n // _BLK
    tail = n - nfull * _BLK

    def body(x_hbm, w_ref, s_ref, o_ref, *scratch):
        bufs = scratch[:_NBUF]
        tbuf = scratch[_NBUF]
        sems = scratch[_NBUF + 1:2 * _NBUF + 1]
        semt = scratch[2 * _NBUF + 1]

        def cp(i):
            return pltpu.make_async_copy(
                x_hbm.at[0, pl.ds(i * _BLK, _BLK), 0, :],
                bufs[i % _NBUF], sems[i % _NBUF])

        def cpt():
            return pltpu.make_async_copy(
                x_hbm.at[0, pl.ds(nfull * _BLK, tail), 0, :], tbuf, semt)

        for i in range(min(_NBUF, nfull)):
            cp(i).start()
        if tail:
            cpt().start()

        w = w_ref[...]
        for i in range(nfull):
            cp(i).wait()
            _epilogue(w, s_ref, bufs[i % _NBUF][...], _BLK, i * _BLK, o_ref)
            if i + _NBUF < nfull:
                cp(i + _NBUF).start()
        if tail:
            cpt().wait()
            _epilogue(w, s_ref, tbuf[...], tail, nfull * _BLK, o_ref)

    return body


@jax.jit
def _run(x, W_cheb, b_cheb, W_time, b_time, W_res, b_res,
         ln_gamma, ln_beta, W_final, b_final):
    _, n, f_in, _ = x.shape
    x4 = x.reshape(x.shape[0], n, 1, f_in)
    w = jnp.zeros((8, f_in), jnp.float32)
    w = w.at[0].set(W_cheb[:, 0]).at[1].set(W_res[0, :, 0, 0])
    scal = jnp.stack([
        b_cheb[0], W_time[0, 0, 0, 1], b_time[0], b_res[0],
        ln_gamma[0], ln_beta[0], W_final[0, 0, 0, 0], b_final[0],
    ])[None, :]

    tail = n - (n // _BLK) * _BLK
    out = pl.pallas_call(
        _make_body(n),
        in_specs=[
            pl.BlockSpec(memory_space=pl.ANY),
            pl.BlockSpec(memory_space=pltpu.MemorySpace.VMEM),
            pl.BlockSpec(memory_space=pltpu.MemorySpace.VMEM),
        ],
        out_specs=pl.BlockSpec(memory_space=pltpu.MemorySpace.VMEM),
        out_shape=jax.ShapeDtypeStruct((1, n), jnp.float32),
        scratch_shapes=(
            [pltpu.VMEM((_BLK, f_in), jnp.float32) for _ in range(_NBUF)]
            + [pltpu.VMEM((max(tail, 8), f_in), jnp.float32)]
            + [pltpu.SemaphoreType.DMA] * (_NBUF + 1)
        ),
    )(x4, w, scal)
    return out.reshape(n, 1)


def kernel(x, edge_index, edge_weight, W_cheb, b_cheb, W_time, b_time,
           W_res, b_res, ln_gamma, ln_beta, W_final, b_final):
    del edge_index, edge_weight
    out = _run(x, W_cheb, b_cheb, W_time, b_time, W_res, b_res,
               ln_gamma, ln_beta, W_final, b_final)
    return (out,)
